# baseline (device time: 14340 ns/iter reference)
import jax
import jax.numpy as jnp
from jax import lax
from jax.experimental import pallas as pl
from jax.experimental.pallas import tpu as pltpu

H = 2
C2 = 2
NC = H * C2


def kernel(t, W):
    m, k = t.shape
    _, n = W.shape
    mh = m // H
    mc = mh // C2

    def body(t_hbm, w_hbm, out_hbm, tv, wv, acc, comm_ref,
             in_sems, w_sem, out_sems, send_sems, recv_sems):
        my = lax.axis_index("i")
        peer1 = my ^ 1
        peer2 = 3 - my
        peers = (peer1, peer2)

        barrier_sem = pltpu.get_barrier_semaphore()
        for nbr in peers:
            pl.semaphore_signal(
                barrier_sem, inc=1,
                device_id=(nbr,), device_id_type=pl.DeviceIdType.MESH,
            )

        chunks = [
            (h, c, h * mh + c * mc) for c in range(C2) for h in range(H)
        ]

        w_cp = pltpu.make_async_copy(w_hbm, wv, w_sem)
        w_cp.start()
        in_cps = []
        for i, (h, c, row) in enumerate(chunks):
            rows = pl.ds(row, mc)
            cp = pltpu.make_async_copy(
                t_hbm.at[rows, :], tv.at[rows, :], in_sems.at[i]
            )
            cp.start()
            in_cps.append(cp)

        def exchange(stage, h, c, row, peer):
            return pltpu.make_async_remote_copy(
                src_ref=acc.at[pl.ds(row, mc), :],
                dst_ref=comm_ref.at[stage, pl.ds(row, mc), :],
                send_sem=send_sems.at[stage, h, c],
                recv_sem=recv_sems.at[stage, h, c],
                device_id=(peer,),
                device_id_type=pl.DeviceIdType.MESH,
            )

        w_cp.wait()
        s1 = {}
        for i, (h, c, row) in enumerate(chunks):
            rows = pl.ds(row, mc)
            in_cps[i].wait()
            acc[rows, :] = jnp.dot(
                tv[rows, :], wv[...], preferred_element_type=jnp.float32
            )
            if i == 0:
                pl.semaphore_wait(barrier_sem, 2)
            r = exchange(0, h, c, row, peers[h])
            r.start()
            s1[(h, c)] = r

        s2 = {}
        for h, c, row in chunks:
            rows = pl.ds(row, mc)
            s1[(h, c)].wait()
            acc[rows, :] += comm_ref[0, rows, :]
            r = exchange(1, h, c, row, peers[1 - h])
            r.start()
            s2[(h, c)] = r

        out_cps = []
        for i, (h, c, row) in enumerate(chunks):
            rows = pl.ds(row, mc)
            s2[(h, c)].wait()
            acc[rows, :] += comm_ref[1, rows, :]
            cp = pltpu.make_async_copy(
                acc.at[rows, :], out_hbm.at[rows, :], out_sems.at[i]
            )
            cp.start()
            out_cps.append(cp)
        for cp in out_cps:
            cp.wait()

    return pl.pallas_call(
        body,
        out_shape=jax.ShapeDtypeStruct((m, n), jnp.float32),
        in_specs=[
            pl.BlockSpec(memory_space=pl.ANY),
            pl.BlockSpec(memory_space=pl.ANY),
        ],
        out_specs=pl.BlockSpec(memory_space=pl.ANY),
        scratch_shapes=[
            pltpu.VMEM((m, k), jnp.float32),
            pltpu.VMEM((k, n), jnp.float32),
            pltpu.VMEM((m, n), jnp.float32),
            pltpu.VMEM((2, m, n), jnp.float32),
            pltpu.SemaphoreType.DMA((NC,)),
            pltpu.SemaphoreType.DMA,
            pltpu.SemaphoreType.DMA((NC,)),
            pltpu.SemaphoreType.DMA((2, H, C2)),
            pltpu.SemaphoreType.DMA((2, H, C2)),
        ],
        compiler_params=pltpu.CompilerParams(collective_id=0),
    )(t, W)


# device time: 13679 ns/iter; 1.0483x vs baseline; 1.0483x over previous
import jax
import jax.numpy as jnp
from jax import lax
from jax.experimental import pallas as pl
from jax.experimental.pallas import tpu as pltpu

H = 2
C2 = 2


def kernel(t, W):
    m, k = t.shape
    _, n = W.shape
    mh = m // H
    mc = mh // C2

    def body(t_ref, w_ref, out_ref, comm_ref, send_sems, recv_sems):
        my = lax.axis_index("i")
        peer1 = my ^ 1
        peer2 = 3 - my
        peers = (peer1, peer2)

        barrier_sem = pltpu.get_barrier_semaphore()
        for nbr in peers:
            pl.semaphore_signal(
                barrier_sem, inc=1,
                device_id=(nbr,), device_id_type=pl.DeviceIdType.MESH,
            )

        chunks = [
            (h, c, h * mh + c * mc) for c in range(C2) for h in range(H)
        ]

        def exchange(stage, h, c, row, peer):
            return pltpu.make_async_remote_copy(
                src_ref=out_ref.at[pl.ds(row, mc), :],
                dst_ref=comm_ref.at[stage, pl.ds(row, mc), :],
                send_sem=send_sems.at[stage, h, c],
                recv_sem=recv_sems.at[stage, h, c],
                device_id=(peer,),
                device_id_type=pl.DeviceIdType.MESH,
            )

        s1 = {}
        for i, (h, c, row) in enumerate(chunks):
            rows = pl.ds(row, mc)
            out_ref[rows, :] = jnp.dot(
                t_ref[rows, :], w_ref[...], preferred_element_type=jnp.float32
            )
            if i == 0:
                pl.semaphore_wait(barrier_sem, 2)
            r = exchange(0, h, c, row, peers[h])
            r.start()
            s1[(h, c)] = r

        s2 = {}
        for h, c, row in chunks:
            rows = pl.ds(row, mc)
            s1[(h, c)].wait()
            out_ref[rows, :] += comm_ref[0, rows, :]
            r = exchange(1, h, c, row, peers[1 - h])
            r.start()
            s2[(h, c)] = r

        for h, c, row in chunks:
            rows = pl.ds(row, mc)
            s2[(h, c)].wait()
            out_ref[rows, :] += comm_ref[1, rows, :]

    return pl.pallas_call(
        body,
        out_shape=jax.ShapeDtypeStruct((m, n), jnp.float32),
        in_specs=[
            pl.BlockSpec(memory_space=pltpu.VMEM),
            pl.BlockSpec(memory_space=pltpu.VMEM),
        ],
        out_specs=pl.BlockSpec(memory_space=pltpu.VMEM),
        scratch_shapes=[
            pltpu.VMEM((2, m, n), jnp.float32),
            pltpu.SemaphoreType.DMA((2, H, C2)),
            pltpu.SemaphoreType.DMA((2, H, C2)),
        ],
        compiler_params=pltpu.CompilerParams(collective_id=0),
    )(t, W)


# device time: 13657 ns/iter; 1.0500x vs baseline; 1.0016x over previous
import jax
import jax.numpy as jnp
from jax import lax
from jax.experimental import pallas as pl
from jax.experimental.pallas import tpu as pltpu

H = 2
C2 = 2


def kernel(t, W):
    m, k = t.shape
    _, n = W.shape
    mh = m // H
    mc = mh // C2

    def body(t_ref, w_ref, out_ref, comm_ref, send_sems, recv_sems):
        my = lax.axis_index("i")
        peer1 = my ^ 1
        peer2 = 3 - my
        peers = (peer1, peer2)

        barrier_sem = pltpu.get_barrier_semaphore()
        for nbr in peers:
            pl.semaphore_signal(
                barrier_sem, inc=1,
                device_id=(nbr,), device_id_type=pl.DeviceIdType.MESH,
            )

        chunks = [
            (h, c, h * mh + c * mc) for c in range(C2) for h in range(H)
        ]

        def exchange(stage, h, c, row, peer):
            return pltpu.make_async_remote_copy(
                src_ref=out_ref.at[pl.ds(row, mc), :],
                dst_ref=comm_ref.at[stage, pl.ds(row, mc), :],
                send_sem=send_sems.at[stage, h, c],
                recv_sem=recv_sems.at[stage, h, c],
                device_id=(peer,),
                device_id_type=pl.DeviceIdType.MESH,
            )

        s1 = {}
        for i, (h, c, row) in enumerate(chunks):
            rows = pl.ds(row, mc)
            out_ref[rows, :] = jnp.dot(
                t_ref[rows, :], w_ref[...], preferred_element_type=jnp.float32
            )
            if i == 0:
                pl.semaphore_wait(barrier_sem, 2)
            r = exchange(0, h, c, row, peers[h])
            r.start()
            s1[(h, c)] = r

        s2 = {}
        for h, c, row in chunks:
            rows = pl.ds(row, mc)
            s1[(h, c)].wait_recv()
            comm_ref[0, rows, :] += out_ref[rows, :]
            r = pltpu.make_async_remote_copy(
                src_ref=comm_ref.at[0, pl.ds(row, mc), :],
                dst_ref=comm_ref.at[1, pl.ds(row, mc), :],
                send_sem=send_sems.at[1, h, c],
                recv_sem=recv_sems.at[1, h, c],
                device_id=(peers[1 - h],),
                device_id_type=pl.DeviceIdType.MESH,
            )
            r.start()
            s2[(h, c)] = r

        for h, c, row in chunks:
            rows = pl.ds(row, mc)
            s2[(h, c)].wait_recv()
            s1[(h, c)].wait_send()
            out_ref[rows, :] = comm_ref[0, rows, :] + comm_ref[1, rows, :]
        for h, c, row in chunks:
            s2[(h, c)].wait_send()

    return pl.pallas_call(
        body,
        out_shape=jax.ShapeDtypeStruct((m, n), jnp.float32),
        in_specs=[
            pl.BlockSpec(memory_space=pltpu.VMEM),
            pl.BlockSpec(memory_space=pltpu.VMEM),
        ],
        out_specs=pl.BlockSpec(memory_space=pltpu.VMEM),
        scratch_shapes=[
            pltpu.VMEM((2, m, n), jnp.float32),
            pltpu.SemaphoreType.DMA((2, H, C2)),
            pltpu.SemaphoreType.DMA((2, H, C2)),
        ],
        compiler_params=pltpu.CompilerParams(collective_id=0),
    )(t, W)
